# Initial kernel scaffold; baseline (speedup 1.0000x reference)
import jax
import jax.numpy as jnp
from jax.experimental import pallas as pl


def kernel(x, he_index, he_attr, he_mark, he_count, batch, response_idx, node_pos, params):
    raise NotImplementedError("write your pallas kernel here")


# speedup vs baseline: 3.2617x; 3.2617x over previous
import functools

import jax
import jax.numpy as jnp
from jax import lax
from jax.experimental import pallas as pl
from jax.experimental.pallas import tpu as pltpu
from jax.experimental.pallas import tpu_sc as plsc

N = 10000
E = 320000
H = 10000
D = 128
HD = 128
ED = 16
G = 8
L = 3
HEADS = 4
DH = HD // HEADS

NC = 2
NS = 16
NW = NC * NS
PER_W = E // NW
CH = 80
NCHUNK = PER_W // CH
ROWS_BIG = 640
ROWS_LAST = 400

_HIGHEST = jax.lax.Precision.HIGHEST


def _mm(a, b):
    return jnp.dot(a, b, precision=_HIGHEST, preferred_element_type=jnp.float32)


def _ln_tc(x, g, b):
    m = jnp.mean(x, axis=-1, keepdims=True)
    v = jnp.mean((x - m) ** 2, axis=-1, keepdims=True)
    return (x - m) * jax.lax.rsqrt(v + 1e-5) * g + b


_MESH = None


def _mesh():
    global _MESH
    if _MESH is None:
        _MESH = plsc.VectorSubcoreMesh(core_axis_name="c", subcore_axis_name="s")
    return _MESH


def _zero_rows_128(buf, rows):
    z = jnp.zeros((16,), jnp.float32)

    def body(r, _):
        for j in range(8):
            buf[r, pl.ds(j * 16, 16)] = z
        return 0

    lax.fori_loop(0, rows, body, 0)


def _spmem_zero_2d(acc, buf, sid, h):
    _zero_rows_128(buf, CH)
    for i in range(ROWS_BIG // CH):
        row0 = sid * ROWS_BIG + i * CH

        @pl.when(row0 < h)
        def _():
            pltpu.sync_copy(buf, acc.at[pl.ds(row0, CH)])


def _spmem_out_2d(acc, out_hbm, cid, sid):
    @pl.when(sid < NS - 1)
    def _():
        pltpu.sync_copy(acc.at[pl.ds(sid * ROWS_BIG, ROWS_BIG)],
                        out_hbm.at[cid, pl.ds(sid * ROWS_BIG, ROWS_BIG)])

    @pl.when(sid == NS - 1)
    def _():
        pltpu.sync_copy(acc.at[pl.ds((NS - 1) * ROWS_BIG, ROWS_LAST)],
                        out_hbm.at[cid, pl.ds((NS - 1) * ROWS_BIG, ROWS_LAST)])


NP1 = 10240
RP1 = NP1 // NS


def _sc_counts(nidx, hidx):
    @functools.partial(
        pl.kernel, mesh=_mesh(),
        out_type=[jax.ShapeDtypeStruct((NC, NP1), jnp.float32),
                  jax.ShapeDtypeStruct((NC, NP1), jnp.float32)],
        scratch_types=[
            pltpu.VMEM((CH,), jnp.int32),
            pltpu.VMEM((CH,), jnp.int32),
            pltpu.VMEM((CH,), jnp.float32),
            pltpu.VMEM((RP1,), jnp.float32),
            pltpu.VMEM_SHARED((NP1,), jnp.float32),
            pltpu.VMEM_SHARED((NP1,), jnp.float32),
        ],
    )
    def k(nidx_hbm, hidx_hbm, deg_hbm, hcnt_hbm,
          nv, hv, ones_v, zv, dacc, hacc):
        cid = lax.axis_index("c")
        sid = lax.axis_index("s")
        wid = cid * NS + sid

        one = jnp.ones((16,), jnp.float32)
        zero = jnp.zeros((16,), jnp.float32)

        def initb(i, _):
            ones_v[pl.ds(i * 16, 16)] = one
            return 0

        lax.fori_loop(0, CH // 16, initb, 0)

        def initz(i, _):
            zv[pl.ds(i * 16, 16)] = zero
            return 0

        lax.fori_loop(0, RP1 // 16, initz, 0)

        pltpu.sync_copy(zv, dacc.at[pl.ds(sid * RP1, RP1)])
        pltpu.sync_copy(zv, hacc.at[pl.ds(sid * RP1, RP1)])

        plsc.subcore_barrier()

        def chunk(i, _):
            base = wid * PER_W + i * CH
            pltpu.sync_copy(nidx_hbm.at[pl.ds(base, CH)], nv)
            pltpu.sync_copy(hidx_hbm.at[pl.ds(base, CH)], hv)
            pltpu.sync_copy(ones_v, dacc.at[nv], add=True)
            pltpu.sync_copy(ones_v, hacc.at[hv], add=True)
            return 0

        lax.fori_loop(0, NCHUNK, chunk, 0)
        plsc.subcore_barrier()

        pltpu.sync_copy(dacc.at[pl.ds(sid * RP1, RP1)],
                        deg_hbm.at[cid, pl.ds(sid * RP1, RP1)])
        pltpu.sync_copy(hacc.at[pl.ds(sid * RP1, RP1)],
                        hcnt_hbm.at[cid, pl.ds(sid * RP1, RP1)])

    return k(nidx, hidx)


def _hsum16(x):
    iota = lax.broadcasted_iota(jnp.int32, (16,), 0)
    dnums = lax.GatherDimensionNumbers(offset_dims=(), collapsed_slice_dims=(0,),
                                       start_index_map=(0,))
    for sh in (1, 2, 4, 8):
        idx = jnp.bitwise_xor(iota, sh)
        x = x + lax.gather(x, idx.reshape(16, 1), dnums, (1,),
                           mode=lax.GatherScatterMode.PROMISE_IN_BOUNDS)
    return x


def _newton_rsqrt(x):
    i = lax.bitcast_convert_type(x, jnp.int32)
    i = jnp.int32(0x5F3759DF) - jnp.right_shift(i, 1)
    y = lax.bitcast_convert_type(i, jnp.float32)
    for _ in range(3):
        y = y * (1.5 - 0.5 * x * y * y)
    return y


def _sc_edge_ln(hw, mw, nidx, hidx, gamma, beta):
    @functools.partial(
        pl.kernel, mesh=_mesh(),
        out_type=jax.ShapeDtypeStruct((NC, H, 128), jnp.float32),
        scratch_types=[
            pltpu.VMEM((CH,), jnp.int32),
            pltpu.VMEM((CH,), jnp.int32),
            pltpu.VMEM((CH, 128), jnp.float32),
            pltpu.VMEM((CH, 128), jnp.float32),
            pltpu.VMEM((128,), jnp.float32),
            pltpu.VMEM((128,), jnp.float32),
            pltpu.VMEM_SHARED((H, 128), jnp.float32),
            pltpu.SemaphoreType.DMA,
            pltpu.SemaphoreType.DMA,
        ],
    )
    def k(hw_hbm, mw_hbm, nidx_hbm, hidx_hbm, g_hbm, b_hbm, out_hbm,
          nv, hv, bufa, bufb, gv, bv, acc, sem1, sem2):
        cid = lax.axis_index("c")
        sid = lax.axis_index("s")
        wid = cid * NS + sid

        pltpu.sync_copy(g_hbm, gv)
        pltpu.sync_copy(b_hbm, bv)
        _spmem_zero_2d(acc, bufa, sid, H)
        plsc.subcore_barrier()

        gs = [gv[pl.ds(j * 16, 16)] for j in range(8)]
        bs = [bv[pl.ds(j * 16, 16)] for j in range(8)]

        def chunk(i, _):
            base = wid * PER_W + i * CH
            pltpu.sync_copy(nidx_hbm.at[pl.ds(base, CH)], nv)
            pltpu.sync_copy(hidx_hbm.at[pl.ds(base, CH)], hv)
            cpa = pltpu.async_copy(hw_hbm.at[nv], bufa, sem1)
            cpb = pltpu.async_copy(mw_hbm.at[hv], bufb, sem2)
            cpa.wait()
            cpb.wait()

            def row(r, _):
                xs = []
                for j in range(8):
                    xs.append(bufa[r, pl.ds(j * 16, 16)]
                              + bufb[r, pl.ds(j * 16, 16)])
                t01 = xs[0] + xs[1]
                t23 = xs[2] + xs[3]
                t45 = xs[4] + xs[5]
                t67 = xs[6] + xs[7]
                tot = (t01 + t23) + (t45 + t67)
                q01 = xs[0] * xs[0] + xs[1] * xs[1]
                q23 = xs[2] * xs[2] + xs[3] * xs[3]
                q45 = xs[4] * xs[4] + xs[5] * xs[5]
                q67 = xs[6] * xs[6] + xs[7] * xs[7]
                tot2 = (q01 + q23) + (q45 + q67)
                m = _hsum16(tot) * (1.0 / 128.0)
                m2 = _hsum16(tot2) * (1.0 / 128.0)
                var = jnp.maximum(m2 - m * m, 0.0) + 1e-5
                inv = _newton_rsqrt(var)
                for j in range(8):
                    t = (xs[j] - m) * inv * gs[j] + bs[j]
                    bufa[r, pl.ds(j * 16, 16)] = jnp.maximum(t, 0.0)
                return 0

            lax.fori_loop(0, CH, row, 0)
            pltpu.sync_copy(bufa, acc.at[hv], add=True)
            return 0

        lax.fori_loop(0, NCHUNK, chunk, 0)
        plsc.subcore_barrier()
        _spmem_out_2d(acc, out_hbm, cid, sid)

    return k(hw, mw, nidx, hidx, gamma, beta)


def _sc_gather_scatter(v, hidx, nidx):
    @functools.partial(
        pl.kernel, mesh=_mesh(),
        out_type=jax.ShapeDtypeStruct((NC, N, 128), jnp.float32),
        scratch_types=[
            pltpu.VMEM((CH,), jnp.int32),
            pltpu.VMEM((CH,), jnp.int32),
            pltpu.VMEM((CH, 128), jnp.float32),
            pltpu.VMEM_SHARED((N, 128), jnp.float32),
            pltpu.SemaphoreType.DMA,
        ],
    )
    def k(v_hbm, hidx_hbm, nidx_hbm, out_hbm, hv, nv, buf, acc, sem):
        cid = lax.axis_index("c")
        sid = lax.axis_index("s")
        wid = cid * NS + sid

        _spmem_zero_2d(acc, buf, sid, N)
        plsc.subcore_barrier()

        def chunk(i, _):
            base = wid * PER_W + i * CH
            pltpu.sync_copy(hidx_hbm.at[pl.ds(base, CH)], hv)
            pltpu.sync_copy(nidx_hbm.at[pl.ds(base, CH)], nv)
            pltpu.async_copy(v_hbm.at[hv], buf, sem).wait()
            pltpu.sync_copy(buf, acc.at[nv], add=True)
            return 0

        lax.fori_loop(0, NCHUNK, chunk, 0)
        plsc.subcore_barrier()
        _spmem_out_2d(acc, out_hbm, cid, sid)

    return k(v, hidx, nidx)


BS = 2000


def _tc_call(body, grid, in_specs, out_specs, out_shape, scratch_shapes=()):
    return pl.pallas_call(
        body, grid=grid, in_specs=in_specs, out_specs=out_specs,
        out_shape=out_shape, scratch_shapes=list(scratch_shapes))


def _rowspec(w):
    return pl.BlockSpec((BS, w), lambda i: (i, 0))


def _fullspec(h, w):
    return pl.BlockSpec((h, w), lambda i: (0, 0))


def _tc_input(x, mark, in_wt, in_b, w1h_t, b1, mark_wt):
    def body(x_r, mk_r, iw_r, ib_r, w1_r, b1_r, mw_r, h_o, hw_o, mc_o):
        h = jnp.maximum(_mm(x_r[...], iw_r[...]) + ib_r[...], 0.0)
        h_o[...] = h
        hw_o[...] = _mm(h, w1_r[...]) + b1_r[...]
        mc_o[...] = _mm(mk_r[...], mw_r[...])

    return _tc_call(
        body, (N // BS,),
        [_rowspec(128), _rowspec(2), _fullspec(128, 128), _fullspec(1, 128),
         _fullspec(128, 128), _fullspec(1, 128), _fullspec(2, 128 * L)],
        [_rowspec(128), _rowspec(128), _rowspec(128 * L)],
        [jax.ShapeDtypeStruct((N, 128), jnp.float32),
         jax.ShapeDtypeStruct((N, 128), jnp.float32),
         jax.ShapeDtypeStruct((N, 128 * L), jnp.float32)],
    )(x, mark, in_wt, in_b, w1h_t, b1, mark_wt)


def _tc_hyper(acc0, acc1, hc0, hc1, he_count, he_attr,
              w2_t, b2, w1a_t, w1g_t, b1e, g_e, be_e, w2e_t, b2e):
    def body(a0, a1, c0, c1, hcnt_r, attr_r,
             w2_r, b2_r, wa_r, wg_r, b1_r, g_r, be_r, w2e_r, b2e_r, v_o):
        aggpre = a0[...] + a1[...]
        cnt = c0[...] + c1[...]
        agg = (_mm(aggpre, w2_r[...]) + cnt * b2_r[...]) / (hcnt_r[...] + 1e-6)
        ew = _mm(attr_r[...], wa_r[...]) + _mm(agg, wg_r[...]) + b1_r[...]
        u = jnp.maximum(_ln_tc(ew, g_r[...], be_r[...]), 0.0)
        v_o[...] = jnp.maximum(_mm(u, w2e_r[...]) + b2e_r[...], 0.0)

    return _tc_call(
        body, (H // BS,),
        [_rowspec(128), _rowspec(128), _rowspec(1), _rowspec(1), _rowspec(1),
         _rowspec(ED), _fullspec(128, 128), _fullspec(1, 128),
         _fullspec(ED, 128), _fullspec(128, 128), _fullspec(1, 128),
         _fullspec(1, 128), _fullspec(1, 128), _fullspec(128, 128),
         _fullspec(1, 128)],
        _rowspec(128),
        jax.ShapeDtypeStruct((H, 128), jnp.float32),
    )(acc0, acc1, hc0, hc1, he_count, he_attr,
      w2_t, b2, w1a_t, w1g_t, b1e, g_e, be_e, w2e_t, b2e)


def _tc_node(nacc0, nacc1, d0, d1, h, lo_g, lo_b, wn_t, bn):
    def body(n0, n1, d0_r, d1_r, h_r, g_r, b_r, wn_r, bn_r, h_o, hw_o):
        outpre = (n0[...] + n1[...]) / (d0_r[...] + d1_r[...] + 1e-6)
        out = _ln_tc(outpre, g_r[...], b_r[...])
        hn = h_r[...] + out
        h_o[...] = hn
        hw_o[...] = _mm(hn, wn_r[...]) + bn_r[...]

    return _tc_call(
        body, (N // BS,),
        [_rowspec(128), _rowspec(128), _rowspec(1), _rowspec(1), _rowspec(128),
         _fullspec(1, 128), _fullspec(1, 128), _fullspec(128, 128),
         _fullspec(1, 128)],
        [_rowspec(128), _rowspec(128)],
        [jax.ShapeDtypeStruct((N, 128), jnp.float32),
         jax.ShapeDtypeStruct((N, 128), jnp.float32)],
    )(nacc0, nacc1, d0, d1, h, lo_g, lo_b, wn_t, bn)


def _tc_pma_pre(h, k, seed, wq_t, bq, wv_t, bv, batch2, pos2, resp2):
    scale = 1.0 / (DH ** 0.5)

    def body(h_r, k_r, sd_r, wq_r, bq_r, wv_r, bv_r, bt_r, ps_r, rp_r,
             v_o, lg_o, mk_o):
        v_o[...] = _mm(h_r[...], wv_r[...]) + bv_r[...]
        q = _mm(sd_r[...], wq_r[...]) + bq_r[...]
        kk = k_r[...]
        lgs = []
        for hh in range(HEADS):
            lgs.append(jnp.sum(kk[:, hh * DH:(hh + 1) * DH]
                               * q[:, hh * DH:(hh + 1) * DH],
                               axis=1, keepdims=True) * scale)
        lg_o[...] = jnp.concatenate(lgs, axis=1)
        bt = bt_r[...]
        gid = jax.lax.broadcasted_iota(jnp.int32, (1, G), 1)
        oh = (bt == gid).astype(jnp.int32)
        rnode = jnp.sum(oh * rp_r[...], axis=1, keepdims=True)
        mk_o[...] = (ps_r[...] >= rnode).astype(jnp.float32)

    return _tc_call(
        body, (N // BS,),
        [_rowspec(128), _rowspec(128), _fullspec(1, 128), _fullspec(128, 128),
         _fullspec(1, 128), _fullspec(128, 128), _fullspec(1, 128),
         _rowspec(1), _rowspec(1), _fullspec(1, G)],
        [_rowspec(128), _rowspec(HEADS), _rowspec(1)],
        [jax.ShapeDtypeStruct((N, 128), jnp.float32),
         jax.ShapeDtypeStruct((N, HEADS), jnp.float32),
         jax.ShapeDtypeStruct((N, 1), jnp.float32)],
    )(h, k, seed, wq_t, bq, wv_t, bv, batch2, pos2, resp2)


def _tc_pma_reduce(logits, v, maskf, batch2):
    nsteps = N // BS

    def body(lg_r, v_r, mk_r, bt_r, o_o, c_o, m_s, s_s, o_s, c_s):
        pi = pl.program_id(0)

        @pl.when(pi == 0)
        def _():
            m_s[...] = jnp.full((G, HEADS), -1e30, jnp.float32)
            s_s[...] = jnp.zeros((G, HEADS), jnp.float32)
            o_s[...] = jnp.zeros((G, 128), jnp.float32)
            c_s[...] = jnp.zeros((G, 1), jnp.float32)

        lg = lg_r[...]
        vv = v_r[...]
        mk = mk_r[...]
        bt = bt_r[...]
        m_all = m_s[...]
        s_all = s_s[...]
        o_all = o_s[...]
        c_all = c_s[...]
        m_l, s_l, o_l, c_l = [], [], [], []
        for i in range(G):
            gm = mk * (bt == i).astype(jnp.float32)
            ci = c_all[i:i + 1] + jnp.sum(gm).reshape(1, 1)
            lm = jnp.where(gm > 0, lg, -1e30)
            bm = jnp.max(lm, axis=0, keepdims=True)
            mo = m_all[i:i + 1]
            mn = jnp.maximum(mo, bm)
            al = jnp.exp(mo - mn)
            pp = jnp.where(gm > 0, jnp.exp(lg - mn), 0.0)
            sn = s_all[i:i + 1] * al + jnp.sum(pp, axis=0, keepdims=True)
            al_rep = jnp.repeat(al, DH, axis=1)
            pv = jnp.concatenate(
                [jnp.sum(pp[:, hh:hh + 1] * vv[:, hh * DH:(hh + 1) * DH],
                         axis=0, keepdims=True) for hh in range(HEADS)],
                axis=1)
            on = o_all[i:i + 1] * al_rep + pv
            m_l.append(mn)
            s_l.append(sn)
            o_l.append(on)
            c_l.append(ci)
        m_new = jnp.concatenate(m_l, axis=0)
        s_new = jnp.concatenate(s_l, axis=0)
        o_new = jnp.concatenate(o_l, axis=0)
        c_new = jnp.concatenate(c_l, axis=0)
        m_s[...] = m_new
        s_s[...] = s_new
        o_s[...] = o_new
        c_s[...] = c_new

        @pl.when(pi == nsteps - 1)
        def _():
            s_rep = jnp.repeat(s_new, DH, axis=1)
            o_o[...] = o_new / s_rep
            c_o[...] = c_new

    return _tc_call(
        body, (nsteps,),
        [_rowspec(HEADS), _rowspec(128), _rowspec(1), _rowspec(1)],
        [pl.BlockSpec((G, 128), lambda i: (0, 0)),
         pl.BlockSpec((G, 1), lambda i: (0, 0))],
        [jax.ShapeDtypeStruct((G, 128), jnp.float32),
         jax.ShapeDtypeStruct((G, 1), jnp.float32)],
        scratch_shapes=[pltpu.VMEM((G, HEADS), jnp.float32),
                        pltpu.VMEM((G, HEADS), jnp.float32),
                        pltpu.VMEM((G, 128), jnp.float32),
                        pltpu.VMEM((G, 1), jnp.float32)],
    )(logits, v, maskf, batch2)


def _tc_pma_post(o, cnt, seed, out_wt, out_b, n1_g, n1_b,
                 f1_t, f1_b, f2_t, f2_b, n2_g, n2_b,
                 pw1_t, pb1, pw2_t, pb2):
    def body(o_r, c_r, sd_r, ow_r, ob_r, g1_r, b1_r, f1w_r, f1b_r,
             f2w_r, f2b_r, g2_r, b2_r, pw1_r, pb1_r, pw2_r, pb2_r, z_o):
        o_ = o_r[...]
        c_ = c_r[...]
        o1 = _mm(o_, ow_r[...]) + ob_r[...]
        o1 = _ln_tc(o1 + sd_r[...], g1_r[...], b1_r[...])
        o2 = _mm(jnp.maximum(_mm(o1, f1w_r[...]) + f1b_r[...], 0.0),
                 f2w_r[...]) + f2b_r[...]
        og = _ln_tc(o1 + o2, g2_r[...], b2_r[...])
        og = jnp.where(c_ > 0, og, 0.0)
        hg = og / jnp.sqrt(jnp.maximum(c_, 1.0))
        z = _mm(jnp.maximum(_mm(hg, pw1_r[...]) + pb1_r[...], 0.0),
                pw2_r[...]) + pb2_r[...]
        z_o[...] = z

    full = _fullspec

    return _tc_call(
        body, (1,),
        [full(G, 128), full(G, 1), full(1, 128), full(128, 128), full(1, 128),
         full(1, 128), full(1, 128), full(128, 4 * HD), full(1, 4 * HD),
         full(4 * HD, 128), full(1, 128), full(1, 128), full(1, 128),
         full(128, HD // 2), full(1, HD // 2), full(HD // 2, 1), full(1, 1)],
        full(G, 1),
        jax.ShapeDtypeStruct((G, 1), jnp.float32),
    )(o, cnt, seed, out_wt, out_b, n1_g, n1_b, f1_t, f1_b, f2_t, f2_b,
      n2_g, n2_b, pw1_t, pb1, pw2_t, pb2)


def _row(v):
    return v.reshape(1, -1)


def kernel(x, he_index, he_attr, he_mark, he_count, batch, response_idx,
           node_pos, params):
    nidx = he_index[0]
    hidx = he_index[1]
    layers = params['layers']

    in_wt = params['in_W'].T
    in_b = _row(params['in_b'])
    w1h_t0 = layers[0]['n2e_W1'][:, :HD].T
    b1_0 = _row(layers[0]['n2e_b1'])
    mark_wt = jnp.concatenate([layers[l]['n2e_W1'][:, HD:].T for l in range(L)],
                              axis=1)

    pm = params['pma']
    wq, wk, wv = jnp.split(pm['in_W'], 3, axis=0)
    bq, bk, bv = jnp.split(pm['in_b'], 3)

    h, hw, mcat = _tc_input(x, he_mark, in_wt, in_b, w1h_t0, b1_0, mark_wt)

    deg2, hcnt2 = _sc_counts(nidx, hidx)
    d0 = deg2[0, :N].reshape(N, 1)
    d1 = deg2[1, :N].reshape(N, 1)
    hc0 = hcnt2[0, :H].reshape(H, 1)
    hc1 = hcnt2[1, :H].reshape(H, 1)
    he_count2 = he_count.reshape(H, 1)

    for l in range(L):
        p = layers[l]
        mw = mcat[:, l * 128:(l + 1) * 128]
        acc2 = _sc_edge_ln(hw, mw, nidx, hidx, p['n2e_g'], p['n2e_be'])
        v = _tc_hyper(acc2[0], acc2[1], hc0, hc1, he_count2, he_attr,
                      p['n2e_W2'].T, _row(p['n2e_b2']),
                      p['e2n_W1'][:, :ED].T, p['e2n_W1'][:, ED:].T,
                      _row(p['e2n_b1']), _row(p['e2n_g']), _row(p['e2n_be']),
                      p['e2n_W2'].T, _row(p['e2n_b2']))
        nacc2 = _sc_gather_scatter(v, hidx, nidx)
        if l + 1 < L:
            wn_t = layers[l + 1]['n2e_W1'][:, :HD].T
            bn = _row(layers[l + 1]['n2e_b1'])
        else:
            wn_t = wk.T
            bn = _row(bk)
        h, hw = _tc_node(nacc2[0], nacc2[1], d0, d1, h,
                         _row(p['lo_g']), _row(p['lo_b']), wn_t, bn)

    batch2 = batch.reshape(N, 1)
    pos2 = node_pos.reshape(N, 1)
    resp2 = response_idx.reshape(1, G)
    v_, logits, maskf = _tc_pma_pre(h, hw, pm['seed'], wq.T, _row(bq),
                                    wv.T, _row(bv), batch2, pos2, resp2)
    o, cnt = _tc_pma_reduce(logits, v_, maskf, batch2)
    z = _tc_pma_post(o, cnt, pm['seed'], pm['out_W'].T, _row(pm['out_b']),
                     _row(pm['n1_g']), _row(pm['n1_b']),
                     pm['ffn_W1'].T, _row(pm['ffn_b1']),
                     pm['ffn_W2'].T, _row(pm['ffn_b2']),
                     _row(pm['n2_g']), _row(pm['n2_b']),
                     params['pred']['W1'].T, _row(params['pred']['b1']),
                     params['pred']['W2'].T, _row(params['pred']['b2']))
    return z.reshape(-1)
